# baseline (speedup 1.0000x reference)
import jax
import jax.numpy as jnp
from jax.experimental import pallas as pl

N_NODES = 10000
HEADS1 = 8
HID = 128
OUT_DIM = 128


def _gat_conv(x, src, dst, W, a_s, a_d, b, heads, out_ch, num_nodes, concat):
    h = (x @ W).reshape(num_nodes, heads, out_ch)
    alpha_s = (h * a_s[None, :, :]).sum(-1)
    alpha_d = (h * a_d[None, :, :]).sum(-1)
    e = jax.nn.leaky_relu(alpha_s[src] + alpha_d[dst], negative_slope=0.2)
    emax = jax.ops.segment_max(e, dst, num_segments=num_nodes)
    emax = jnp.where(jnp.isfinite(emax), emax, 0.0)
    ex = jnp.exp(e - emax[dst])
    denom = jax.ops.segment_sum(ex, dst, num_segments=num_nodes)
    alpha = ex / (denom[dst] + 1e-16)
    msg = h[src] * alpha[:, :, None]
    out = jax.ops.segment_sum(msg, dst, num_segments=num_nodes)
    if concat:
        out = out.reshape(num_nodes, heads * out_ch)
    else:
        out = out.mean(axis=1)
    return out + b


def _bias_add_kernel(x_ref, b_ref, o_ref):
    o_ref[...] = x_ref[...] + b_ref[...]


def kernel(x, edge_index, W1, a1_src, a1_dst, b1, W2, a2_src, a2_dst, b2):
    n = x.shape[0]
    loop = jnp.arange(n, dtype=edge_index.dtype)
    src = jnp.concatenate([edge_index[0], loop])
    dst = jnp.concatenate([edge_index[1], loop])
    h = _gat_conv(x, src, dst, W1, a1_src, a1_dst, b1, HEADS1, HID, n, True)
    h = jax.nn.relu(h)
    out = _gat_conv(h, src, dst, W2, a2_src, a2_dst, jnp.zeros_like(b2), 1,
                    OUT_DIM, n, True)
    out = pl.pallas_call(
        _bias_add_kernel,
        out_shape=jax.ShapeDtypeStruct((n, OUT_DIM), jnp.float32),
        grid=(n // 400,),
        in_specs=[
            pl.BlockSpec((400, OUT_DIM), lambda i: (i, 0)),
            pl.BlockSpec((1, OUT_DIM), lambda i: (0, 0)),
        ],
        out_specs=pl.BlockSpec((400, OUT_DIM), lambda i: (i, 0)),
    )(out, b2.reshape(1, OUT_DIM))
    return out


# speedup vs baseline: 14.7167x; 14.7167x over previous
import jax
import jax.numpy as jnp
from jax import lax
from jax.experimental import pallas as pl
from jax.experimental.pallas import tpu as pltpu
from jax.experimental.pallas import tpu_sc as plsc

N_NODES = 10000
IN_DIM = 128
HID = 128
OUT_DIM = 128
HEADS1 = 8

NC = 2
NS = 16
NW = NC * NS
S = 128
E_TOT = 320000 + N_NODES
K = -(-E_TOT // (NW * S))
E_PAD = NW * K * S
NR = 10112
RT = NR // NS
NBLK = 1000
CBLK = 1264


def _l1_body(x_ref, w_ref, am_ref, *outs):
    h = jnp.dot(x_ref[...], w_ref[...], preferred_element_type=jnp.float32)
    for j in range(HEADS1):
        outs[j][...] = h[:, j * HID:(j + 1) * HID]
    outs[HEADS1][...] = jnp.dot(h, am_ref[...],
                                preferred_element_type=jnp.float32)


def _layer1_tc(x, W1, amask):
    grid = (N_NODES // NBLK,)
    hspec = pl.BlockSpec((NBLK, HID), lambda i: (i, 0))
    return pl.pallas_call(
        _l1_body,
        grid=grid,
        in_specs=[
            pl.BlockSpec((NBLK, IN_DIM), lambda i: (i, 0)),
            pl.BlockSpec((IN_DIM, HEADS1 * HID), lambda i: (0, 0)),
            pl.BlockSpec((HEADS1 * HID, 16), lambda i: (0, 0)),
        ],
        out_specs=[hspec] * HEADS1 + [pl.BlockSpec((NBLK, 16), lambda i: (i, 0))],
        out_shape=[jax.ShapeDtypeStruct((N_NODES, HID), jnp.float32)] * HEADS1
        + [jax.ShapeDtypeStruct((N_NODES, 16), jnp.float32)],
    )(x, W1, amask)


def _mid_body(num_ref, den_ref, w2_ref, b1_ref, am2_ref, h2w_ref, al2_ref):
    sel = jnp.tile(jnp.eye(HEADS1, dtype=jnp.float32), (NW, 1))
    dsum = jnp.dot(den_ref[...], sel,
                   preferred_element_type=jnp.float32) + 1e-16
    parts = []
    for j in range(HEADS1):
        blkj = (num_ref[0, j] + num_ref[1, j]) / dsum[:, j][:, None]
        hj = blkj + b1_ref[0, j * HID:(j + 1) * HID]
        parts.append(jnp.maximum(hj, 0.0))
    h2 = jnp.concatenate(parts, axis=-1)
    o = jnp.dot(h2, w2_ref[...], preferred_element_type=jnp.float32)
    h2w_ref[...] = o
    al2_ref[...] = jnp.dot(o, am2_ref[...], preferred_element_type=jnp.float32)


def _mid_tc(num1, den1, W2, b1, am2):
    grid = (NR // CBLK,)
    return pl.pallas_call(
        _mid_body,
        grid=grid,
        in_specs=[
            pl.BlockSpec((NC, HEADS1, CBLK, HID), lambda i: (0, 0, i, 0)),
            pl.BlockSpec((NW, HEADS1, CBLK), lambda i: (0, 0, i)),
            pl.BlockSpec((HEADS1 * HID, OUT_DIM), lambda i: (0, 0)),
            pl.BlockSpec((1, HEADS1 * HID), lambda i: (0, 0)),
            pl.BlockSpec((OUT_DIM, 16), lambda i: (0, 0)),
        ],
        out_specs=[
            pl.BlockSpec((CBLK, OUT_DIM), lambda i: (i, 0)),
            pl.BlockSpec((CBLK, 16), lambda i: (i, 0)),
        ],
        out_shape=[
            jax.ShapeDtypeStruct((NR, OUT_DIM), jnp.float32),
            jax.ShapeDtypeStruct((NR, 16), jnp.float32),
        ],
    )(num1, den1, W2, b1, am2)


def _fin_body(num_ref, den_ref, b2_ref, o_ref):
    dsum = jnp.sum(den_ref[...], axis=1) + 1e-16
    o_ref[...] = (num_ref[0, 0] + num_ref[1, 0]) / dsum[:, None] + b2_ref[0]


def _final_tc(num2, den2, b2):
    grid = (NR // CBLK,)
    return pl.pallas_call(
        _fin_body,
        grid=grid,
        in_specs=[
            pl.BlockSpec((NC, 1, CBLK, HID), lambda i: (0, 0, i, 0)),
            pl.BlockSpec((CBLK, NW), lambda i: (i, 0)),
            pl.BlockSpec((1, OUT_DIM), lambda i: (0, 0)),
        ],
        out_specs=pl.BlockSpec((CBLK, OUT_DIM), lambda i: (i, 0)),
        out_shape=jax.ShapeDtypeStruct((NR, OUT_DIM), jnp.float32),
    )(num2, den2, b2)


def _edge_call(nh, src2d, dst2d, alphaT, zr, z1, h_list):
    mesh = plsc.VectorSubcoreMesh(core_axis_name="c", subcore_axis_name="s")
    scratch = [
        pltpu.VMEM((S,), jnp.int32),
        pltpu.VMEM((S,), jnp.int32),
        pltpu.VMEM((S, HID), jnp.float32),
        pltpu.VMEM((S,), jnp.float32),
        pltpu.VMEM((NR,), jnp.float32),
        pltpu.VMEM((NR,), jnp.float32),
        pltpu.VMEM((NR,), jnp.float32),
        pltpu.VMEM((16,), jnp.int32),
        pltpu.VMEM((16,), jnp.float32),
        pltpu.VMEM_SHARED((NR, HID), jnp.float32),
        pltpu.SemaphoreType.DMA,
    ]

    def body(src_hbm, dst_hbm, at_hbm, zr_hbm, z1_hbm, *rest):
        h_hbms = rest[:nh]
        num_hbm, den_hbm = rest[nh], rest[nh + 1]
        (srcv, dstv, g, exb, asj, adj, dloc, ib16, fb16,
         accum, sem) = rest[nh + 2:]
        cid = lax.axis_index("c")
        sid = lax.axis_index("s")
        wid = sid * NC + cid
        lanes16 = lax.iota(jnp.int32, 16)

        for j in range(nh):
            pltpu.sync_copy(at_hbm.at[j], asj)
            pltpu.sync_copy(at_hbm.at[nh + j], adj)
            pltpu.sync_copy(zr_hbm, accum.at[pl.ds(sid * RT, RT)])
            pltpu.sync_copy(z1_hbm, dloc)
            plsc.subcore_barrier()

            def chunk(k, _):
                ci = pltpu.async_copy(
                    src_hbm.at[wid, pl.ds(k * S, S)], srcv, sem)
                cj = pltpu.async_copy(
                    dst_hbm.at[wid, pl.ds(k * S, S)], dstv, sem)
                ci.wait()
                cj.wait()
                cg = pltpu.async_copy(h_hbms[j].at[srcv], g, sem)

                def lanes(t, _):
                    si = srcv[pl.ds(t * 16, 16)]
                    di = dstv[pl.ds(t * 16, 16)]
                    e = (plsc.load_gather(asj, [si])
                         + plsc.load_gather(adj, [di]))
                    e = jnp.where(e > 0, e, 0.2 * e)
                    ex = jnp.exp(e)
                    exb[pl.ds(t * 16, 16)] = ex

                    d_s, ex_s = plsc.sort_key_val(di, ex)
                    cs = plsc.cumsum(ex_s)
                    ib16[pl.ds(0, 16)] = d_s
                    fb16[pl.ds(0, 16)] = cs
                    nxt = plsc.load_gather(
                        ib16, [jnp.minimum(lanes16 + 1, 15)])
                    is_last = jnp.logical_or(d_s != nxt, lanes16 == 15)
                    prv = plsc.load_gather(
                        ib16, [jnp.maximum(lanes16 - 1, 0)])
                    is_first = jnp.logical_or(d_s != prv, lanes16 == 0)
                    run0 = plsc.cummax(jnp.where(is_first, lanes16, 0))
                    csp = plsc.load_gather(
                        fb16, [jnp.maximum(run0 - 1, 0)])
                    csp = jnp.where(run0 == 0, jnp.float32(0.0), csp)
                    plsc.addupdate_scatter(dloc, [d_s], cs - csp,
                                           mask=is_last)
                    return 0
                lax.fori_loop(0, S // 16, lanes, 0)
                cg.wait()

                def rows(r, _):
                    ev = plsc.load_gather(
                        exb, [jnp.full((16,), 0, jnp.int32) + r])
                    for c in range(8):
                        g[r, pl.ds(c * 16, 16)] = (
                            g[r, pl.ds(c * 16, 16)] * ev)
                    return 0
                lax.fori_loop(0, S, rows, 0)

                pltpu.sync_copy(g, accum.at[dstv], add=True)
                return 0
            lax.fori_loop(0, K, chunk, 0)
            plsc.subcore_barrier()

            pltpu.sync_copy(accum.at[pl.ds(sid * RT, RT)],
                            num_hbm.at[cid, j, pl.ds(sid * RT, RT)])
            pltpu.sync_copy(dloc, den_hbm.at[wid, j])

    fn = pl.kernel(
        body, mesh=mesh,
        out_type=(
            jax.ShapeDtypeStruct((NC, nh, NR, HID), jnp.float32),
            jax.ShapeDtypeStruct((NW, nh, NR), jnp.float32),
        ),
        scratch_types=scratch,
        compiler_params=pltpu.CompilerParams(needs_layout_passes=False))
    return fn(src2d, dst2d, alphaT, zr, z1, *h_list)


@jax.jit
def kernel(x, edge_index, W1, a1_src, a1_dst, b1, W2, a2_src, a2_dst, b2):
    n = x.shape[0]
    loop = jnp.arange(n, dtype=jnp.int32)
    src = jnp.concatenate([edge_index[0].astype(jnp.int32), loop,
                           jnp.zeros((E_PAD - E_TOT,), jnp.int32)])
    dst = jnp.concatenate([edge_index[1].astype(jnp.int32), loop,
                           jnp.full((E_PAD - E_TOT,), N_NODES, jnp.int32)])
    src2d = src.reshape(NW, K * S)
    dst2d = dst.reshape(NW, K * S)
    zr = jnp.zeros((RT, HID), jnp.float32)
    z1 = jnp.zeros((NR,), jnp.float32)

    eye8 = jnp.eye(HEADS1, dtype=jnp.float32)
    am_s = (a1_src[:, :, None] * eye8[:, None, :]).reshape(HEADS1 * HID, HEADS1)
    am_d = (a1_dst[:, :, None] * eye8[:, None, :]).reshape(HEADS1 * HID, HEADS1)
    amask = jnp.concatenate([am_s, am_d], axis=1)
    am2 = jnp.zeros((OUT_DIM, 16), jnp.float32)
    am2 = am2.at[:, 0].set(a2_src[0]).at[:, 1].set(a2_dst[0])

    outs = _layer1_tc(x, W1, amask)
    h_heads, alpha1 = outs[:HEADS1], outs[HEADS1]
    at1 = jnp.pad(alpha1, ((0, NR - n), (0, 0))).T

    num1, den1 = _edge_call(HEADS1, src2d, dst2d, at1, zr, z1, list(h_heads))
    den1t = den1.reshape(NW * HEADS1, NR).T

    h2w, alpha2 = _mid_tc(num1, den1t, W2, b1.reshape(1, -1), am2)
    at2 = alpha2[:, :2].at[n:].set(0.0).T

    num2, den2 = _edge_call(1, src2d, dst2d, at2, zr, z1, [h2w])
    den2t = den2.reshape(NW, NR).T

    out = _final_tc(num2, den2t, b2.reshape(1, -1))
    return out[:n]


# speedup vs baseline: 16.1168x; 1.0951x over previous
import jax
import jax.numpy as jnp
from jax import lax
from jax.experimental import pallas as pl
from jax.experimental.pallas import tpu as pltpu
from jax.experimental.pallas import tpu_sc as plsc

N_NODES = 10000
IN_DIM = 128
HID = 128
OUT_DIM = 128
HEADS1 = 8

NC = 2
NS = 16
NW = NC * NS
S = 128
E_TOT = 320000 + N_NODES
K = -(-E_TOT // (NW * S))
E_PAD = NW * K * S
NR = 10112
RT = NR // NS
NBLK = 1000
CBLK = 1264


def _l1_body(x_ref, w_ref, am_ref, *outs):
    h = jnp.dot(x_ref[...], w_ref[...], preferred_element_type=jnp.float32)
    for j in range(HEADS1):
        outs[j][...] = h[:, j * HID:(j + 1) * HID]
    outs[HEADS1][...] = jnp.dot(h, am_ref[...],
                                preferred_element_type=jnp.float32)


def _layer1_tc(x, W1, amask):
    grid = (N_NODES // NBLK,)
    hspec = pl.BlockSpec((NBLK, HID), lambda i: (i, 0))
    return pl.pallas_call(
        _l1_body,
        grid=grid,
        in_specs=[
            pl.BlockSpec((NBLK, IN_DIM), lambda i: (i, 0)),
            pl.BlockSpec((IN_DIM, HEADS1 * HID), lambda i: (0, 0)),
            pl.BlockSpec((HEADS1 * HID, 16), lambda i: (0, 0)),
        ],
        out_specs=[hspec] * HEADS1 + [pl.BlockSpec((NBLK, 16), lambda i: (i, 0))],
        out_shape=[jax.ShapeDtypeStruct((N_NODES, HID), jnp.float32)] * HEADS1
        + [jax.ShapeDtypeStruct((N_NODES, 16), jnp.float32)],
    )(x, W1, amask)


def _mid_body(num_ref, den_ref, w2_ref, b1_ref, am2_ref, h2w_ref, al2_ref):
    sel = jnp.tile(jnp.eye(HEADS1, dtype=jnp.float32), (NW, 1))
    dsum = jnp.dot(den_ref[...], sel,
                   preferred_element_type=jnp.float32) + 1e-16
    parts = []
    for j in range(HEADS1):
        blkj = (num_ref[0, j] + num_ref[1, j]) / dsum[:, j][:, None]
        hj = blkj + b1_ref[0, j * HID:(j + 1) * HID]
        parts.append(jnp.maximum(hj, 0.0))
    h2 = jnp.concatenate(parts, axis=-1)
    o = jnp.dot(h2, w2_ref[...], preferred_element_type=jnp.float32)
    h2w_ref[...] = o
    al2_ref[...] = jnp.dot(o, am2_ref[...], preferred_element_type=jnp.float32)


def _mid_tc(num1, den1, W2, b1, am2):
    grid = (NR // CBLK,)
    return pl.pallas_call(
        _mid_body,
        grid=grid,
        in_specs=[
            pl.BlockSpec((NC, HEADS1, CBLK, HID), lambda i: (0, 0, i, 0)),
            pl.BlockSpec((NW, HEADS1, CBLK), lambda i: (0, 0, i)),
            pl.BlockSpec((HEADS1 * HID, OUT_DIM), lambda i: (0, 0)),
            pl.BlockSpec((1, HEADS1 * HID), lambda i: (0, 0)),
            pl.BlockSpec((OUT_DIM, 16), lambda i: (0, 0)),
        ],
        out_specs=[
            pl.BlockSpec((CBLK, OUT_DIM), lambda i: (i, 0)),
            pl.BlockSpec((CBLK, 16), lambda i: (i, 0)),
        ],
        out_shape=[
            jax.ShapeDtypeStruct((NR, OUT_DIM), jnp.float32),
            jax.ShapeDtypeStruct((NR, 16), jnp.float32),
        ],
    )(num1, den1, W2, b1, am2)


def _fin_body(num_ref, den_ref, b2_ref, o_ref):
    dsum = jnp.sum(den_ref[...], axis=1) + 1e-16
    o_ref[...] = (num_ref[0, 0] + num_ref[1, 0]) / dsum[:, None] + b2_ref[0]


def _final_tc(num2, den2, b2):
    grid = (NR // CBLK,)
    return pl.pallas_call(
        _fin_body,
        grid=grid,
        in_specs=[
            pl.BlockSpec((NC, 1, CBLK, HID), lambda i: (0, 0, i, 0)),
            pl.BlockSpec((CBLK, NW), lambda i: (i, 0)),
            pl.BlockSpec((1, OUT_DIM), lambda i: (0, 0)),
        ],
        out_specs=pl.BlockSpec((CBLK, OUT_DIM), lambda i: (i, 0)),
        out_shape=jax.ShapeDtypeStruct((NR, OUT_DIM), jnp.float32),
    )(num2, den2, b2)


def _edge_call(nh, src2d, dst2d, alphaT, zr, z1, h_list):
    mesh = plsc.VectorSubcoreMesh(core_axis_name="c", subcore_axis_name="s")
    scratch = [
        pltpu.VMEM((2 * S,), jnp.int32),
        pltpu.VMEM((2 * S,), jnp.int32),
        pltpu.VMEM((S, HID), jnp.float32),
        pltpu.VMEM((S,), jnp.float32),
        pltpu.VMEM((NR,), jnp.float32),
        pltpu.VMEM((NR,), jnp.float32),
        pltpu.VMEM((NR,), jnp.float32),
        pltpu.VMEM((16,), jnp.int32),
        pltpu.VMEM((16,), jnp.float32),
        pltpu.VMEM_SHARED((NR, HID), jnp.float32),
        pltpu.SemaphoreType.DMA,
        pltpu.SemaphoreType.DMA,
    ]

    def body(src_hbm, dst_hbm, at_hbm, zr_hbm, z1_hbm, *rest):
        h_hbms = rest[:nh]
        num_hbm, den_hbm = rest[nh], rest[nh + 1]
        (srcv, dstv, g, exb, asj, adj, dloc, ib16, fb16,
         accum, sem, sem2) = rest[nh + 2:]
        cid = lax.axis_index("c")
        sid = lax.axis_index("s")
        wid = sid * NC + cid
        lanes16 = lax.iota(jnp.int32, 16)

        for j in range(nh):
            pltpu.sync_copy(at_hbm.at[j], asj)
            pltpu.sync_copy(at_hbm.at[nh + j], adj)
            pltpu.sync_copy(zr_hbm, accum.at[pl.ds(sid * RT, RT)])
            pltpu.sync_copy(z1_hbm, dloc)
            pltpu.async_copy(src_hbm.at[wid, pl.ds(0, S)],
                             srcv.at[pl.ds(0, S)], sem2)
            pltpu.async_copy(dst_hbm.at[wid, pl.ds(0, S)],
                             dstv.at[pl.ds(0, S)], sem2)
            plsc.subcore_barrier()

            def chunk(k, _):
                cur = pl.multiple_of(lax.rem(k, 2) * S, S)
                nxt = pl.multiple_of(lax.rem(k + 1, 2) * S, S)
                nk = lax.rem(k + 1, K)
                pltpu.make_async_copy(src_hbm.at[wid, pl.ds(0, S)],
                                      srcv.at[pl.ds(0, S)], sem2).wait()
                pltpu.make_async_copy(dst_hbm.at[wid, pl.ds(0, S)],
                                      dstv.at[pl.ds(0, S)], sem2).wait()
                pltpu.async_copy(src_hbm.at[wid, pl.ds(nk * S, S)],
                                 srcv.at[pl.ds(nxt, S)], sem2)
                pltpu.async_copy(dst_hbm.at[wid, pl.ds(nk * S, S)],
                                 dstv.at[pl.ds(nxt, S)], sem2)
                cg = pltpu.async_copy(
                    h_hbms[j].at[srcv.at[pl.ds(cur, S)]], g, sem)

                def lanes(t, _):
                    si = srcv[pl.ds(cur + t * 16, 16)]
                    di = dstv[pl.ds(cur + t * 16, 16)]
                    e = (plsc.load_gather(asj, [si])
                         + plsc.load_gather(adj, [di]))
                    e = jnp.where(e > 0, e, 0.2 * e)
                    ex = jnp.exp(e)
                    exb[pl.ds(t * 16, 16)] = ex

                    d_s, ex_s = plsc.sort_key_val(di, ex)
                    cs = plsc.cumsum(ex_s)
                    ib16[pl.ds(0, 16)] = d_s
                    fb16[pl.ds(0, 16)] = cs
                    nxt = plsc.load_gather(
                        ib16, [jnp.minimum(lanes16 + 1, 15)])
                    is_last = jnp.logical_or(d_s != nxt, lanes16 == 15)
                    prv = plsc.load_gather(
                        ib16, [jnp.maximum(lanes16 - 1, 0)])
                    is_first = jnp.logical_or(d_s != prv, lanes16 == 0)
                    run0 = plsc.cummax(jnp.where(is_first, lanes16, 0))
                    csp = plsc.load_gather(
                        fb16, [jnp.maximum(run0 - 1, 0)])
                    csp = jnp.where(run0 == 0, jnp.float32(0.0), csp)
                    plsc.addupdate_scatter(dloc, [d_s], cs - csp,
                                           mask=is_last)
                    return 0
                lax.fori_loop(0, S // 16, lanes, 0)
                cg.wait()

                def rows(r, _):
                    ev = plsc.load_gather(
                        exb, [jnp.full((16,), 0, jnp.int32) + r])
                    for c in range(8):
                        g[r, pl.ds(c * 16, 16)] = (
                            g[r, pl.ds(c * 16, 16)] * ev)
                    return 0
                lax.fori_loop(0, S, rows, 0)

                pltpu.sync_copy(g, accum.at[dstv.at[pl.ds(cur, S)]],
                                add=True)
                return 0
            lax.fori_loop(0, K, chunk, 0)
            pltpu.make_async_copy(src_hbm.at[wid, pl.ds(0, S)],
                                  srcv.at[pl.ds(0, S)], sem2).wait()
            pltpu.make_async_copy(dst_hbm.at[wid, pl.ds(0, S)],
                                  dstv.at[pl.ds(0, S)], sem2).wait()
            plsc.subcore_barrier()

            pltpu.sync_copy(accum.at[pl.ds(sid * RT, RT)],
                            num_hbm.at[cid, j, pl.ds(sid * RT, RT)])
            pltpu.sync_copy(dloc, den_hbm.at[wid, j])

    fn = pl.kernel(
        body, mesh=mesh,
        out_type=(
            jax.ShapeDtypeStruct((NC, nh, NR, HID), jnp.float32),
            jax.ShapeDtypeStruct((NW, nh, NR), jnp.float32),
        ),
        scratch_types=scratch,
        compiler_params=pltpu.CompilerParams(needs_layout_passes=False))
    return fn(src2d, dst2d, alphaT, zr, z1, *h_list)


@jax.jit
def kernel(x, edge_index, W1, a1_src, a1_dst, b1, W2, a2_src, a2_dst, b2):
    n = x.shape[0]
    loop = jnp.arange(n, dtype=jnp.int32)
    src = jnp.concatenate([edge_index[0].astype(jnp.int32), loop,
                           jnp.zeros((E_PAD - E_TOT,), jnp.int32)])
    dst = jnp.concatenate([edge_index[1].astype(jnp.int32), loop,
                           jnp.full((E_PAD - E_TOT,), N_NODES, jnp.int32)])
    src2d = src.reshape(NW, K * S)
    dst2d = dst.reshape(NW, K * S)
    zr = jnp.zeros((RT, HID), jnp.float32)
    z1 = jnp.zeros((NR,), jnp.float32)

    eye8 = jnp.eye(HEADS1, dtype=jnp.float32)
    am_s = (a1_src[:, :, None] * eye8[:, None, :]).reshape(HEADS1 * HID, HEADS1)
    am_d = (a1_dst[:, :, None] * eye8[:, None, :]).reshape(HEADS1 * HID, HEADS1)
    amask = jnp.concatenate([am_s, am_d], axis=1)
    am2 = jnp.zeros((OUT_DIM, 16), jnp.float32)
    am2 = am2.at[:, 0].set(a2_src[0]).at[:, 1].set(a2_dst[0])

    outs = _layer1_tc(x, W1, amask)
    h_heads, alpha1 = outs[:HEADS1], outs[HEADS1]
    at1 = jnp.pad(alpha1, ((0, NR - n), (0, 0))).T

    num1, den1 = _edge_call(HEADS1, src2d, dst2d, at1, zr, z1, list(h_heads))
    den1t = den1.reshape(NW * HEADS1, NR).T

    h2w, alpha2 = _mid_tc(num1, den1t, W2, b1.reshape(1, -1), am2)
    at2 = alpha2[:, :2].at[n:].set(0.0).T

    num2, den2 = _edge_call(1, src2d, dst2d, at2, zr, z1, [h2w])
    den2t = den2.reshape(NW, NR).T

    out = _final_tc(num2, den2t, b2.reshape(1, -1))
    return out[:n]


# speedup vs baseline: 17.2003x; 1.0672x over previous
import jax
import jax.numpy as jnp
from jax import lax
from jax.experimental import pallas as pl
from jax.experimental.pallas import tpu as pltpu
from jax.experimental.pallas import tpu_sc as plsc

N_NODES = 10000
IN_DIM = 128
HID = 128
OUT_DIM = 128
HEADS1 = 8

NC = 2
NS = 16
NW = NC * NS
S = 128
E_TOT = 320000 + N_NODES
K = -(-E_TOT // (NW * S))
E_PAD = NW * K * S
NR = 10112
RT = NR // NS
NBLK = 1000
CBLK = 1264


def _l1_body(x_ref, w_ref, am_ref, *outs):
    h = jnp.dot(x_ref[...], w_ref[...], preferred_element_type=jnp.float32)
    for j in range(HEADS1):
        outs[j][...] = h[:, j * HID:(j + 1) * HID]
    outs[HEADS1][...] = jnp.dot(h, am_ref[...],
                                preferred_element_type=jnp.float32)


def _layer1_tc(x, W1, amask):
    grid = (N_NODES // NBLK,)
    hspec = pl.BlockSpec((NBLK, HID), lambda i: (i, 0))
    return pl.pallas_call(
        _l1_body,
        grid=grid,
        in_specs=[
            pl.BlockSpec((NBLK, IN_DIM), lambda i: (i, 0)),
            pl.BlockSpec((IN_DIM, HEADS1 * HID), lambda i: (0, 0)),
            pl.BlockSpec((HEADS1 * HID, 16), lambda i: (0, 0)),
        ],
        out_specs=[hspec] * HEADS1 + [pl.BlockSpec((NBLK, 16), lambda i: (i, 0))],
        out_shape=[jax.ShapeDtypeStruct((N_NODES, HID), jnp.float32)] * HEADS1
        + [jax.ShapeDtypeStruct((N_NODES, 16), jnp.float32)],
    )(x, W1, amask)


def _mid_body(num_ref, den_ref, w2_ref, b1_ref, am2_ref, h2w_ref, al2_ref):
    sel = jnp.tile(jnp.eye(HEADS1, dtype=jnp.float32), (NW, 1))
    dsum = jnp.dot(den_ref[...], sel,
                   preferred_element_type=jnp.float32) + 1e-16
    parts = []
    for j in range(HEADS1):
        blkj = (num_ref[0, j] + num_ref[1, j]) / dsum[:, j][:, None]
        hj = blkj + b1_ref[0, j * HID:(j + 1) * HID]
        parts.append(jnp.maximum(hj, 0.0))
    h2 = jnp.concatenate(parts, axis=-1)
    o = jnp.dot(h2, w2_ref[...], preferred_element_type=jnp.float32)
    h2w_ref[...] = o
    al2_ref[...] = jnp.dot(o, am2_ref[...], preferred_element_type=jnp.float32)


def _mid_tc(num1, den1, W2, b1, am2):
    grid = (NR // CBLK,)
    return pl.pallas_call(
        _mid_body,
        grid=grid,
        in_specs=[
            pl.BlockSpec((NC, HEADS1, CBLK, HID), lambda i: (0, 0, i, 0)),
            pl.BlockSpec((CBLK, NW * HEADS1), lambda i: (i, 0)),
            pl.BlockSpec((HEADS1 * HID, OUT_DIM), lambda i: (0, 0)),
            pl.BlockSpec((1, HEADS1 * HID), lambda i: (0, 0)),
            pl.BlockSpec((OUT_DIM, 16), lambda i: (0, 0)),
        ],
        out_specs=[
            pl.BlockSpec((CBLK, OUT_DIM), lambda i: (i, 0)),
            pl.BlockSpec((CBLK, 16), lambda i: (i, 0)),
        ],
        out_shape=[
            jax.ShapeDtypeStruct((NR, OUT_DIM), jnp.float32),
            jax.ShapeDtypeStruct((NR, 16), jnp.float32),
        ],
    )(num1, den1, W2, b1, am2)


def _fin_body(num_ref, den_ref, b2_ref, o_ref):
    dsum = jnp.sum(den_ref[...], axis=1) + 1e-16
    o_ref[...] = (num_ref[0, 0] + num_ref[1, 0]) / dsum[:, None] + b2_ref[0]


def _final_tc(num2, den2, b2):
    grid = (NR // CBLK,)
    return pl.pallas_call(
        _fin_body,
        grid=grid,
        in_specs=[
            pl.BlockSpec((NC, 1, CBLK, HID), lambda i: (0, 0, i, 0)),
            pl.BlockSpec((CBLK, NW), lambda i: (i, 0)),
            pl.BlockSpec((1, OUT_DIM), lambda i: (0, 0)),
        ],
        out_specs=pl.BlockSpec((CBLK, OUT_DIM), lambda i: (i, 0)),
        out_shape=jax.ShapeDtypeStruct((NR, OUT_DIM), jnp.float32),
    )(num2, den2, b2)


def _edge_call(nh, src2d, dst2d, alphaT, zr, z1, h_list):
    mesh = plsc.VectorSubcoreMesh(core_axis_name="c", subcore_axis_name="s")
    scratch = [
        pltpu.VMEM((2 * S,), jnp.int32),
        pltpu.VMEM((2 * S,), jnp.int32),
        pltpu.VMEM((S, HID), jnp.float32),
        pltpu.VMEM((S,), jnp.float32),
        pltpu.VMEM((NR,), jnp.float32),
        pltpu.VMEM((NR,), jnp.float32),
        pltpu.VMEM((NR,), jnp.float32),
        pltpu.VMEM((16,), jnp.int32),
        pltpu.VMEM((16,), jnp.float32),
        pltpu.VMEM_SHARED((NR, HID), jnp.float32),
        pltpu.SemaphoreType.DMA,
        pltpu.SemaphoreType.DMA,
        pltpu.SemaphoreType.DMA,
    ]

    def body(src_hbm, dst_hbm, at_hbm, zr_hbm, z1_hbm, *rest):
        h_hbms = rest[:nh]
        num_hbm, den_hbm = rest[nh], rest[nh + 1]
        (srcv, dstv, g, exb, asj, adj, dloc, ib16, fb16,
         accum, sem, sem2, sem3) = rest[nh + 2:]
        cid = lax.axis_index("c")
        sid = lax.axis_index("s")
        wid = sid * NC + cid
        lanes16 = lax.iota(jnp.int32, 16)

        for j in range(nh):
            pltpu.sync_copy(at_hbm.at[j], asj)
            pltpu.sync_copy(at_hbm.at[nh + j], adj)
            pltpu.sync_copy(zr_hbm, accum.at[pl.ds(sid * RT, RT)])
            pltpu.sync_copy(z1_hbm, dloc)
            pltpu.async_copy(src_hbm.at[wid, pl.ds(0, S)],
                             srcv.at[pl.ds(0, S)], sem2)
            pltpu.async_copy(dst_hbm.at[wid, pl.ds(0, S)],
                             dstv.at[pl.ds(0, S)], sem2)
            plsc.subcore_barrier()

            def chunk(k, _):
                cur = pl.multiple_of(lax.rem(k, 2) * S, S)
                nxt = pl.multiple_of(lax.rem(k + 1, 2) * S, S)
                nk = lax.rem(k + 1, K)
                pltpu.make_async_copy(src_hbm.at[wid, pl.ds(0, S)],
                                      srcv.at[pl.ds(0, S)], sem2).wait()
                pltpu.make_async_copy(dst_hbm.at[wid, pl.ds(0, S)],
                                      dstv.at[pl.ds(0, S)], sem2).wait()
                pltpu.async_copy(src_hbm.at[wid, pl.ds(nk * S, S)],
                                 srcv.at[pl.ds(nxt, S)], sem2)
                pltpu.async_copy(dst_hbm.at[wid, pl.ds(nk * S, S)],
                                 dstv.at[pl.ds(nxt, S)], sem2)

                @pl.when(k > 0)
                def _drain_scatter():
                    for _ in range(2):
                        pltpu.make_async_copy(
                            zr_hbm.at[pl.ds(0, S // 2)],
                            g.at[pl.ds(0, S // 2)], sem3).wait()

                cg = pltpu.async_copy(
                    h_hbms[j].at[srcv.at[pl.ds(cur, S)]], g, sem)

                def lanes(t, _):
                    si = srcv[pl.ds(cur + t * 16, 16)]
                    di = dstv[pl.ds(cur + t * 16, 16)]
                    e = (plsc.load_gather(asj, [si])
                         + plsc.load_gather(adj, [di]))
                    e = jnp.where(e > 0, e, 0.2 * e)
                    ex = jnp.exp(e)
                    exb[pl.ds(t * 16, 16)] = ex

                    d_s, ex_s = plsc.sort_key_val(di, ex)
                    cs = plsc.cumsum(ex_s)
                    ib16[pl.ds(0, 16)] = d_s
                    fb16[pl.ds(0, 16)] = cs
                    nxt = plsc.load_gather(
                        ib16, [jnp.minimum(lanes16 + 1, 15)])
                    is_last = jnp.logical_or(d_s != nxt, lanes16 == 15)
                    prv = plsc.load_gather(
                        ib16, [jnp.maximum(lanes16 - 1, 0)])
                    is_first = jnp.logical_or(d_s != prv, lanes16 == 0)
                    run0 = plsc.cummax(jnp.where(is_first, lanes16, 0))
                    csp = plsc.load_gather(
                        fb16, [jnp.maximum(run0 - 1, 0)])
                    csp = jnp.where(run0 == 0, jnp.float32(0.0), csp)
                    plsc.addupdate_scatter(dloc, [d_s], cs - csp,
                                           mask=is_last)
                    return 0
                lax.fori_loop(0, S // 16, lanes, 0)
                cg.wait()

                def rows(r, _):
                    ev = plsc.load_gather(
                        exb, [jnp.full((16,), 0, jnp.int32) + r])
                    for c in range(8):
                        g[r, pl.ds(c * 16, 16)] = (
                            g[r, pl.ds(c * 16, 16)] * ev)
                    return 0
                lax.fori_loop(0, S // 2, rows, 0)
                pltpu.async_copy(
                    g.at[pl.ds(0, S // 2)],
                    accum.at[dstv.at[pl.ds(cur, S // 2)]], sem3, add=True)
                lax.fori_loop(S // 2, S, rows, 0)
                pltpu.async_copy(
                    g.at[pl.ds(S // 2, S // 2)],
                    accum.at[dstv.at[pl.ds(cur + S // 2, S // 2)]],
                    sem3, add=True)
                return 0
            lax.fori_loop(0, K, chunk, 0)
            for _ in range(2):
                pltpu.make_async_copy(zr_hbm.at[pl.ds(0, S // 2)],
                                      g.at[pl.ds(0, S // 2)], sem3).wait()
            pltpu.make_async_copy(src_hbm.at[wid, pl.ds(0, S)],
                                  srcv.at[pl.ds(0, S)], sem2).wait()
            pltpu.make_async_copy(dst_hbm.at[wid, pl.ds(0, S)],
                                  dstv.at[pl.ds(0, S)], sem2).wait()
            plsc.subcore_barrier()

            pltpu.sync_copy(accum.at[pl.ds(sid * RT, RT)],
                            num_hbm.at[cid, j, pl.ds(sid * RT, RT)])
            pltpu.sync_copy(dloc, den_hbm.at[wid, j])

    fn = pl.kernel(
        body, mesh=mesh,
        out_type=(
            jax.ShapeDtypeStruct((NC, nh, NR, HID), jnp.float32),
            jax.ShapeDtypeStruct((NW, nh, NR), jnp.float32),
        ),
        scratch_types=scratch,
        compiler_params=pltpu.CompilerParams(needs_layout_passes=False))
    return fn(src2d, dst2d, alphaT, zr, z1, *h_list)


@jax.jit
def kernel(x, edge_index, W1, a1_src, a1_dst, b1, W2, a2_src, a2_dst, b2):
    n = x.shape[0]
    loop = jnp.arange(n, dtype=jnp.int32)
    src = jnp.concatenate([edge_index[0].astype(jnp.int32), loop,
                           jnp.zeros((E_PAD - E_TOT,), jnp.int32)])
    dst = jnp.concatenate([edge_index[1].astype(jnp.int32), loop,
                           jnp.full((E_PAD - E_TOT,), N_NODES, jnp.int32)])
    src2d = src.reshape(NW, K * S)
    dst2d = dst.reshape(NW, K * S)
    zr = jnp.zeros((RT, HID), jnp.float32)
    z1 = jnp.zeros((NR,), jnp.float32)

    eye8 = jnp.eye(HEADS1, dtype=jnp.float32)
    am_s = (a1_src[:, :, None] * eye8[:, None, :]).reshape(HEADS1 * HID, HEADS1)
    am_d = (a1_dst[:, :, None] * eye8[:, None, :]).reshape(HEADS1 * HID, HEADS1)
    amask = jnp.concatenate([am_s, am_d], axis=1)
    am2 = jnp.zeros((OUT_DIM, 16), jnp.float32)
    am2 = am2.at[:, 0].set(a2_src[0]).at[:, 1].set(a2_dst[0])

    outs = _layer1_tc(x, W1, amask)
    h_heads, alpha1 = outs[:HEADS1], outs[HEADS1]
    at1 = jnp.pad(alpha1, ((0, NR - n), (0, 0))).T

    num1, den1 = _edge_call(HEADS1, src2d, dst2d, at1, zr, z1, list(h_heads))
    den1t = den1.reshape(NW * HEADS1, NR).T

    h2w, alpha2 = _mid_tc(num1, den1t, W2, b1.reshape(1, -1), am2)
    at2 = alpha2[:, :2].at[n:].set(0.0).T

    num2, den2 = _edge_call(1, src2d, dst2d, at2, zr, z1, [h2w])
    den2t = den2.reshape(NW, NR).T

    out = _final_tc(num2, den2t, b2.reshape(1, -1))
    return out[:n]
